# baseline (speedup 1.0000x reference)
import functools

import jax
import jax.numpy as jnp
from jax import lax
from jax.experimental import pallas as pl
from jax.experimental.pallas import tpu as pltpu
from jax.experimental.pallas import tpu_sc as plsc

_B, _L, _V, _E, _A = 1024, 200, 100000, 128, 8
_NC, _NS = 2, 16
_NW = _NC * _NS
_N = _B * _L
_TPW = _N // _NW
_STEP = 128
_NSTEP = _TPW // _STEP


def _body(seq_hbm, a0_hbm, a1_hbm, pidx_hbm,
          tok_hbm, pos_hbm, at0_hbm, at1_hbm,
          out_hbm,
          seq_v, a0_v, a1_v, pidx_v, rows0, rows1, sem_g0, sem_g1, sem_o):
  c = lax.axis_index("c")
  s = lax.axis_index("s")
  wid = s * _NC + c

  pltpu.sync_copy(seq_hbm.at[wid], seq_v)
  pltpu.sync_copy(a0_hbm.at[wid], a0_v)
  pltpu.sync_copy(a1_hbm.at[wid], a1_v)
  pltpu.sync_copy(pidx_hbm.at[wid], pidx_v)

  def adds(j, rows, sem):
    d0 = pltpu.async_copy(at0_hbm.at[a0_v.at[j]], rows, sem, add=True)
    d1 = pltpu.async_copy(at1_hbm.at[a1_v.at[j]], rows, sem, add=True)
    dp = pltpu.async_copy(pos_hbm.at[pidx_v.at[j]], rows, sem, add=True)
    return d0, d1, dp

  def out_slice(j):
    return out_hbm.at[pl.ds(wid * _TPW + j * _STEP, _STEP)]

  def step(jj, carry):
    j0 = jj
    j1 = jj + 1

    @pl.when(jj > 0)
    def _drain():
      pltpu.make_async_copy(rows0, out_slice(0), sem_o).wait()
      pltpu.make_async_copy(rows1, out_slice(0), sem_o).wait()

    t0 = pltpu.async_copy(tok_hbm.at[seq_v.at[j0]], rows0, sem_g0)
    t1 = pltpu.async_copy(tok_hbm.at[seq_v.at[j1]], rows1, sem_g1)
    t0.wait()
    a00, a01, a0p = adds(j0, rows0, sem_g0)
    t1.wait()
    a10, a11, a1p = adds(j1, rows1, sem_g1)
    a00.wait(); a01.wait(); a0p.wait()
    pltpu.async_copy(rows0, out_slice(j0), sem_o)
    a10.wait(); a11.wait(); a1p.wait()
    pltpu.async_copy(rows1, out_slice(j1), sem_o)
    return carry

  lax.fori_loop(0, _NSTEP // 2, lambda i, carry: step(i * 2, carry), 0)
  pltpu.make_async_copy(rows0, out_slice(0), sem_o).wait()
  pltpu.make_async_copy(rows1, out_slice(0), sem_o).wait()


@jax.jit
def _embed(seq, a0, a1, pidx, token_table, pos_table, attr_table0, attr_table1):
  mesh = plsc.VectorSubcoreMesh(core_axis_name="c", subcore_axis_name="s")
  return pl.kernel(
      _body,
      out_type=jax.ShapeDtypeStruct((_N, _E), jnp.float32),
      mesh=mesh,
      scratch_types=[
          pltpu.VMEM((_NSTEP, _STEP), jnp.int32),
          pltpu.VMEM((_NSTEP, _STEP), jnp.int32),
          pltpu.VMEM((_NSTEP, _STEP), jnp.int32),
          pltpu.VMEM((_NSTEP, _STEP), jnp.int32),
          pltpu.VMEM((_STEP, _E), jnp.float32),
          pltpu.VMEM((_STEP, _E), jnp.float32),
          pltpu.SemaphoreType.DMA,
          pltpu.SemaphoreType.DMA,
          pltpu.SemaphoreType.DMA,
      ],
  )(seq, a0, a1, pidx, token_table, pos_table, attr_table0, attr_table1)


def kernel(sequence, attrs_idxs, token_table, pos_table, attr_table0, attr_table1):
  seq = sequence.astype(jnp.int32).reshape(_NW, _NSTEP, _STEP)
  a0 = attrs_idxs[0].astype(jnp.int32).reshape(_NW, _NSTEP, _STEP)
  a1 = attrs_idxs[1].astype(jnp.int32).reshape(_NW, _NSTEP, _STEP)
  pidx = jnp.broadcast_to(
      jnp.arange(_L, dtype=jnp.int32), (_B, _L)).reshape(_NW, _NSTEP, _STEP)
  out = _embed(seq, a0, a1, pidx,
               token_table, pos_table, attr_table0, attr_table1)
  return out.reshape(_B, _L, _E)


# speedup vs baseline: 4.1834x; 4.1834x over previous
import functools

import jax
import jax.numpy as jnp
from jax import lax
from jax.experimental import pallas as pl
from jax.experimental.pallas import tpu as pltpu
from jax.experimental.pallas import tpu_sc as plsc

_B, _L, _V, _E, _A = 1024, 200, 100000, 128, 8
_NC, _NS = 2, 16
_NW = _NC * _NS
_N = _B * _L
_TPW = _N // _NW
_STEP = 128
_NSTEP = _TPW // _STEP
_NK = _E // 16


def _body(seq_hbm, a0_hbm, a1_hbm,
          tok_hbm, pos_hbm, at0_hbm, at1_hbm,
          out_hbm,
          seq_v, cid_v, tmp_v, pos_v, at0_v, at1_v, combo_v,
          rows0, rows1, sem_g0, sem_g1, sem_o0, sem_o1):
  c = lax.axis_index("c")
  s = lax.axis_index("s")
  wid = s * _NC + c

  pltpu.sync_copy(seq_hbm.at[wid], seq_v)
  pltpu.sync_copy(a0_hbm.at[wid], cid_v)
  pltpu.sync_copy(a1_hbm.at[wid], tmp_v)
  pltpu.sync_copy(pos_hbm, pos_v)
  pltpu.sync_copy(at0_hbm, at0_v)
  pltpu.sync_copy(at1_hbm, at1_v)

  pltpu.async_copy(tok_hbm.at[seq_v.at[0]], rows0, sem_g0)

  def cid_row(j, carry):
    for k in range(_STEP // 16):
      sl = pl.ds(k * 16, 16)
      cid_v[j, sl] = cid_v[j, sl] * 8 + tmp_v[j, sl]
    return carry
  lax.fori_loop(0, _NSTEP, cid_row, 0)

  def combo_row(r, carry):
    r0 = r // 8
    r1 = lax.rem(r, 8)
    for k in range(_NK):
      sl = pl.ds(k * 16, 16)
      combo_v[r, sl] = at0_v[r0, sl] + at1_v[r1, sl]
    return carry
  lax.fori_loop(0, _A * _A, combo_row, 0)

  def out_slice(j):
    return out_hbm.at[pl.ds(wid * _TPW + j * _STEP, _STEP)]

  def gather(j, rows, sem):
    pltpu.async_copy(tok_hbm.at[seq_v.at[j]], rows, sem)

  def wait_gather(rows, sem):
    pltpu.make_async_copy(tok_hbm.at[seq_v.at[0]], rows, sem).wait()

  def wait_out(rows, sem):
    pltpu.make_async_copy(rows, out_slice(0), sem).wait()

  def accum(j, rows):
    def grp(g, carry):
      base = g * 16
      cvec = cid_v[j, pl.ds(base, 16)]
      for t in range(16):
        ci = cvec[t]
        i = base + t
        l = lax.rem(j * _STEP + i, _L)
        for k in range(_NK):
          sl = pl.ds(k * 16, 16)
          plsc.addupdate(rows.at[i, sl], pos_v[l, sl] + combo_v[ci, sl])
      return carry
    lax.fori_loop(0, _STEP // 16, grp, 0)

  def pair(jj, carry):
    wait_gather(rows0, sem_g0)
    @pl.when(jj > 0)
    def _drain1():
      wait_out(rows1, sem_o1)
    gather(jj + 1, rows1, sem_g1)
    accum(jj, rows0)
    pltpu.async_copy(rows0, out_slice(jj), sem_o0)
    wait_gather(rows1, sem_g1)
    wait_out(rows0, sem_o0)
    @pl.when(jj + 2 < _NSTEP)
    def _next():
      gather(jj + 2, rows0, sem_g0)
    accum(jj + 1, rows1)
    pltpu.async_copy(rows1, out_slice(jj + 1), sem_o1)
    return carry

  lax.fori_loop(0, _NSTEP // 2, lambda i, carry: pair(i * 2, carry), 0)
  wait_out(rows1, sem_o1)


@jax.jit
def _embed(seq, a0, a1, token_table, pos_table, attr_table0, attr_table1):
  mesh = plsc.VectorSubcoreMesh(core_axis_name="c", subcore_axis_name="s")
  return pl.kernel(
      _body,
      out_type=jax.ShapeDtypeStruct((_N, _E), jnp.float32),
      mesh=mesh,
      scratch_types=[
          pltpu.VMEM((_NSTEP, _STEP), jnp.int32),
          pltpu.VMEM((_NSTEP, _STEP), jnp.int32),
          pltpu.VMEM((_NSTEP, _STEP), jnp.int32),
          pltpu.VMEM((_L, _E), jnp.float32),
          pltpu.VMEM((_A, _E), jnp.float32),
          pltpu.VMEM((_A, _E), jnp.float32),
          pltpu.VMEM((_A * _A, _E), jnp.float32),
          pltpu.VMEM((_STEP, _E), jnp.float32),
          pltpu.VMEM((_STEP, _E), jnp.float32),
          pltpu.SemaphoreType.DMA,
          pltpu.SemaphoreType.DMA,
          pltpu.SemaphoreType.DMA,
          pltpu.SemaphoreType.DMA,
      ],
  )(seq, a0, a1, token_table, pos_table, attr_table0, attr_table1)


def kernel(sequence, attrs_idxs, token_table, pos_table, attr_table0, attr_table1):
  seq = sequence.astype(jnp.int32).reshape(_NW, _NSTEP, _STEP)
  a0 = attrs_idxs[0].astype(jnp.int32).reshape(_NW, _NSTEP, _STEP)
  a1 = attrs_idxs[1].astype(jnp.int32).reshape(_NW, _NSTEP, _STEP)
  out = _embed(seq, a0, a1, token_table, pos_table, attr_table0, attr_table1)
  return out.reshape(_B, _L, _E)


# speedup vs baseline: 6.4089x; 1.5320x over previous
import functools

import jax
import jax.numpy as jnp
from jax import lax
from jax.experimental import pallas as pl
from jax.experimental.pallas import tpu as pltpu
from jax.experimental.pallas import tpu_sc as plsc

_B, _L, _V, _E, _A = 1024, 200, 100000, 128, 8
_NC, _NS = 2, 16
_NW = _NC * _NS
_N = _B * _L
_TPW = _N // _NW
_STEP = _L
_NSTEP = _TPW // _STEP
_HALF = _STEP // 2
_NK = _E // 16
_NG = 12


def _body(seq_hbm, a0_hbm, a1_hbm,
          tok_hbm, pos_hbm, at0_hbm, at1_hbm,
          out_hbm,
          seq_v, a0_v, a1_v, cid_v, pos_v, at0_v, at1_v, combo_v,
          rows0, rows1, sem_g0, sem_g1, sem_o0, sem_o1):
  c = lax.axis_index("c")
  s = lax.axis_index("s")
  wid = s * _NC + c

  pltpu.sync_copy(seq_hbm.at[wid], seq_v)
  pltpu.async_copy(tok_hbm.at[seq_v.at[0]], rows0.at[pl.ds(0, _HALF)], sem_g0)
  pltpu.async_copy(tok_hbm.at[seq_v.at[1]], rows0.at[pl.ds(_HALF, _HALF)],
                   sem_g0)
  pltpu.sync_copy(a0_hbm.at[wid], a0_v)
  pltpu.sync_copy(a1_hbm.at[wid], a1_v)
  pltpu.sync_copy(pos_hbm, pos_v)
  pltpu.sync_copy(at0_hbm, at0_v)
  pltpu.sync_copy(at1_hbm, at1_v)

  def cid_row(r, carry):
    for k in range(_NG + 1):
      sl = pl.ds(min(k * 16, _STEP - 16), 16)
      cid_v[r, sl] = a0_v[r, sl] * 8 + a1_v[r, sl]
    return carry
  lax.fori_loop(0, _NSTEP, cid_row, 0)

  def combo_row(r, carry):
    r0 = r // 8
    r1 = lax.rem(r, 8)
    for k in range(_NK):
      sl = pl.ds(k * 16, 16)
      combo_v[r, sl] = at0_v[r0, sl] + at1_v[r1, sl]
    return carry
  lax.fori_loop(0, _A * _A, combo_row, 0)

  def out_slice(j):
    return out_hbm.at[pl.ds(wid * _TPW + j * _STEP, _STEP)]

  def gather(j, rows, sem):
    pltpu.async_copy(tok_hbm.at[seq_v.at[2 * j]],
                     rows.at[pl.ds(0, _HALF)], sem)
    pltpu.async_copy(tok_hbm.at[seq_v.at[2 * j + 1]],
                     rows.at[pl.ds(_HALF, _HALF)], sem)

  def wait_gather(rows, sem):
    pltpu.make_async_copy(tok_hbm.at[seq_v.at[0]],
                          rows.at[pl.ds(0, _HALF)], sem).wait()
    pltpu.make_async_copy(tok_hbm.at[seq_v.at[0]],
                          rows.at[pl.ds(_HALF, _HALF)], sem).wait()

  def wait_out(rows, sem):
    pltpu.make_async_copy(rows, out_slice(0), sem).wait()

  def accum(j, rows):
    @plsc.parallel_loop(0, _STEP, 1, unroll=2)
    def _pos(i):
      for k in range(_NK):
        sl = pl.ds(k * 16, 16)
        plsc.addupdate(rows.at[i, sl], pos_v[i, sl])

    def cgrp16(i0, cvec, lo):
      for t in range(lo, 16):
        ci = cvec[t]
        i = i0 + t
        for k in range(_NK):
          sl = pl.ds(k * 16, 16)
          plsc.addupdate(rows.at[i, sl], combo_v[ci, sl])

    @plsc.parallel_loop(0, _NG, 1, unroll=1)
    def _combo(g):
      cgrp16(g * 16, cid_v[j, pl.ds(g * 16, 16)], 0)
    cgrp16(_STEP - 16, cid_v[j, pl.ds(_STEP - 16, 16)], 8)

  def pair(jj, carry):
    wait_gather(rows0, sem_g0)
    @pl.when(jj > 0)
    def _drain1():
      wait_out(rows1, sem_o1)
    gather(jj + 1, rows1, sem_g1)
    accum(jj, rows0)
    pltpu.async_copy(rows0, out_slice(jj), sem_o0)
    wait_gather(rows1, sem_g1)
    wait_out(rows0, sem_o0)
    @pl.when(jj + 2 < _NSTEP)
    def _next():
      gather(jj + 2, rows0, sem_g0)
    accum(jj + 1, rows1)
    pltpu.async_copy(rows1, out_slice(jj + 1), sem_o1)
    return carry

  lax.fori_loop(0, _NSTEP // 2, lambda i, carry: pair(i * 2, carry), 0)
  wait_out(rows1, sem_o1)


@jax.jit
def _embed(seq, a0, a1, token_table, pos_table, attr_table0, attr_table1):
  mesh = plsc.VectorSubcoreMesh(core_axis_name="c", subcore_axis_name="s")
  return pl.kernel(
      _body,
      out_type=jax.ShapeDtypeStruct((_N, _E), jnp.float32),
      mesh=mesh,
      scratch_types=[
          pltpu.VMEM((2 * _NSTEP, _HALF), jnp.int32),
          pltpu.VMEM((_NSTEP, _STEP), jnp.int32),
          pltpu.VMEM((_NSTEP, _STEP), jnp.int32),
          pltpu.VMEM((_NSTEP, _STEP), jnp.int32),
          pltpu.VMEM((_L, _E), jnp.float32),
          pltpu.VMEM((_A, _E), jnp.float32),
          pltpu.VMEM((_A, _E), jnp.float32),
          pltpu.VMEM((_A * _A, _E), jnp.float32),
          pltpu.VMEM((_STEP, _E), jnp.float32),
          pltpu.VMEM((_STEP, _E), jnp.float32),
          pltpu.SemaphoreType.DMA,
          pltpu.SemaphoreType.DMA,
          pltpu.SemaphoreType.DMA,
          pltpu.SemaphoreType.DMA,
      ],
  )(seq, a0, a1, token_table, pos_table, attr_table0, attr_table1)


def kernel(sequence, attrs_idxs, token_table, pos_table, attr_table0, attr_table1):
  seq = sequence.astype(jnp.int32).reshape(_NW, 2 * _NSTEP, _HALF)
  a0 = attrs_idxs[0].astype(jnp.int32).reshape(_NW, _NSTEP, _STEP)
  a1 = attrs_idxs[1].astype(jnp.int32).reshape(_NW, _NSTEP, _STEP)
  out = _embed(seq, a0, a1, token_table, pos_table, attr_table0, attr_table1)
  return out.reshape(_B, _L, _E)
